# baseline (speedup 1.0000x reference)
import functools

import numpy as np
import jax
import jax.numpy as jnp
from jax import lax
from jax.experimental import pallas as pl
from jax.experimental.pallas import tpu as pltpu
from jax.experimental.pallas import tpu_sc as plsc

BOUND = 1.0
MIN_NEAR = 0.2
DENSITY_THRESH = 0.01
MAX_STEPS = 128
GRID_SIZE = 128
N_RAYS = 16384
DT = 2.0 * BOUND * float(np.sqrt(3.0)) / MAX_STEPS
NCELLS = GRID_SIZE ** 3
NWORDS = NCELLS // 32

NC, NS = 2, 16
NW = NC * NS
RPW = N_RAYS // NW
CELLS_PW = NCELLS // NW
WORDS_PW = NWORDS // NW


def _spread7(v: int) -> int:
    r = 0
    for i in range(7):
        r |= ((v >> i) & 1) << (3 * i)
    return r


_SP = np.array([_spread7(v) for v in range(GRID_SIZE)], dtype=np.int32)
_SP_TABLES = (_SP, _SP << 1, _SP << 2)

def _wid():
    return lax.axis_index("s") * NC + lax.axis_index("c")


def _pack_body(density_hbm, bitmap_hbm, vals_ref, words_ref):
    wid = _wid()
    pltpu.sync_copy(density_hbm.at[pl.ds(wid * CELLS_PW, CELLS_PW)], vals_ref)
    iota = lax.iota(jnp.int32, 16)

    def wv_body(wv, carry):
        base_idx = iota * 32 + wv * 512
        word = jnp.zeros((16,), jnp.int32)
        for b in range(32):
            v = plsc.load_gather(vals_ref, [base_idx + b])
            word = word | ((v > DENSITY_THRESH).astype(jnp.int32) << b)
        words_ref[pl.ds(wv * 16, 16)] = word
        return carry

    lax.fori_loop(0, WORDS_PW // 16, wv_body, 0)
    pltpu.sync_copy(words_ref, bitmap_hbm.at[pl.ds(wid * WORDS_PW, WORDS_PW)])


def _march_body(rays_hbm, bitmap_hbm, sp0_hbm, sp1_hbm, sp2_hbm,
                xyz_hbm, z_hbm, valid_hbm,
                occ_ref, rays_ref, sp0_ref, sp1_ref, sp2_ref,
                xyz_st, z_st, valid_st):
    wid = _wid()
    ray0 = wid * RPW
    pltpu.sync_copy(bitmap_hbm, occ_ref)
    pltpu.sync_copy(rays_hbm.at[pl.ds(ray0 * 6, RPW * 6)], rays_ref)
    pltpu.sync_copy(sp0_hbm, sp0_ref)
    pltpu.sync_copy(sp1_hbm, sp1_ref)
    pltpu.sync_copy(sp2_hbm, sp2_ref)

    iota = lax.iota(jnp.int32, 16)
    iota6 = iota * 6
    iota128 = iota * MAX_STEPS
    iota384 = iota * (MAX_STEPS * 3)

    def chunk_body(c, carry):
        i6 = iota6 + c * 96
        dx = plsc.load_gather(rays_ref, [i6])
        dy = plsc.load_gather(rays_ref, [i6 + 1])
        dz = plsc.load_gather(rays_ref, [i6 + 2])
        ox = plsc.load_gather(rays_ref, [i6 + 3])
        oy = plsc.load_gather(rays_ref, [i6 + 4])
        oz = plsc.load_gather(rays_ref, [i6 + 5])

        def inv(d):
            safe = jnp.where(jnp.abs(d) > 1e-9, d, jnp.float32(1e-9))
            return 1.0 / safe

        ivx, ivy, ivz = inv(dx), inv(dy), inv(dz)
        t0x = (-BOUND - ox) * ivx
        t1x = (BOUND - ox) * ivx
        t0y = (-BOUND - oy) * ivy
        t1y = (BOUND - oy) * ivy
        t0z = (-BOUND - oz) * ivz
        t1z = (BOUND - oz) * ivz
        tmin = jnp.maximum(jnp.maximum(jnp.minimum(t0x, t1x),
                                       jnp.minimum(t0y, t1y)),
                           jnp.minimum(t0z, t1z))
        tmax = jnp.minimum(jnp.minimum(jnp.maximum(t0x, t1x),
                                       jnp.maximum(t0y, t1y)),
                           jnp.maximum(t0z, t1z))
        nears = jnp.maximum(tmin, MIN_NEAR)
        hit = tmax > jnp.maximum(tmin, 0.0)
        fars = jnp.where(hit, jnp.maximum(tmax, nears), nears)

        def step_body(s, inner):
            sf = s.astype(jnp.float32)
            t = nears + (sf + 0.5) * DT
            in_range = t < fars
            x = ox + t * dx
            y = oy + t * dy
            z = oz + t * dz

            def coord(u):
                uc = jnp.clip(u, -BOUND, BOUND)
                cf = (uc * 0.5 + 0.5) * float(GRID_SIZE)
                return jnp.clip(cf.astype(jnp.int32), 0, GRID_SIZE - 1)

            code = (plsc.load_gather(sp0_ref, [coord(x)])
                    | plsc.load_gather(sp1_ref, [coord(y)])
                    | plsc.load_gather(sp2_ref, [coord(z)]))
            wrd = plsc.load_gather(occ_ref, [code >> 5])
            occ = (lax.shift_right_logical(wrd, code & 31) & 1) != 0
            valid = in_range & occ

            ia = iota384 + s * 3
            plsc.store_scatter(xyz_st, [ia], jnp.where(valid, x, 0.0))
            plsc.store_scatter(xyz_st, [ia + 1], jnp.where(valid, y, 0.0))
            plsc.store_scatter(xyz_st, [ia + 2], jnp.where(valid, z, 0.0))
            iz = iota128 + s
            plsc.store_scatter(z_st, [iz], jnp.where(valid, t, 0.0))
            plsc.store_scatter(valid_st, [iz], valid.astype(jnp.int32))
            return inner

        lax.fori_loop(0, MAX_STEPS, step_body, 0)

        obase = ray0 + c * 16
        pltpu.sync_copy(xyz_st, xyz_hbm.at[pl.ds(obase * 384, 16 * 384)])
        pltpu.sync_copy(z_st, z_hbm.at[pl.ds(obase * 128, 16 * 128)])
        pltpu.sync_copy(valid_st, valid_hbm.at[pl.ds(obase * 128, 16 * 128)])
        return carry

    lax.fori_loop(0, RPW // 16, chunk_body, 0)


@functools.cache
def _build():
    mesh = plsc.VectorSubcoreMesh(
        core_axis_name="c", subcore_axis_name="s",
        num_cores=NC, num_subcores=NS,
    )
    params = pltpu.CompilerParams(
        needs_layout_passes=False, use_tc_tiling_on_sc=False
    )
    pack = pl.kernel(
        _pack_body,
        out_type=jax.ShapeDtypeStruct((NWORDS,), jnp.int32),
        mesh=mesh,
        compiler_params=params,
        scratch_types=[
            pltpu.VMEM((CELLS_PW,), jnp.float32),
            pltpu.VMEM((WORDS_PW,), jnp.int32),
        ],
    )
    march = pl.kernel(
        _march_body,
        out_type=[
            jax.ShapeDtypeStruct((N_RAYS * MAX_STEPS * 3,), jnp.float32),
            jax.ShapeDtypeStruct((N_RAYS * MAX_STEPS,), jnp.float32),
            jax.ShapeDtypeStruct((N_RAYS * MAX_STEPS,), jnp.int32),
        ],
        mesh=mesh,
        compiler_params=params,
        scratch_types=[
            pltpu.VMEM((NWORDS,), jnp.int32),
            pltpu.VMEM((RPW * 6,), jnp.float32),
            pltpu.VMEM((GRID_SIZE,), jnp.int32),
            pltpu.VMEM((GRID_SIZE,), jnp.int32),
            pltpu.VMEM((GRID_SIZE,), jnp.int32),
            pltpu.VMEM((16 * MAX_STEPS * 3,), jnp.float32),
            pltpu.VMEM((16 * MAX_STEPS,), jnp.float32),
            pltpu.VMEM((16 * MAX_STEPS,), jnp.int32),
        ],
    )
    return pack, march


def kernel(rays_chunk, focal, density_grid):
    del focal
    rays_flat = rays_chunk.reshape(-1)
    dens_flat = density_grid.reshape(-1)
    sp0 = jnp.asarray(_SP_TABLES[0])
    sp1 = jnp.asarray(_SP_TABLES[1])
    sp2 = jnp.asarray(_SP_TABLES[2])
    pack, march = _build()
    bitmap = pack(dens_flat)
    xyz, z, valid = march(rays_flat, bitmap, sp0, sp1, sp2)
    return (xyz.reshape(N_RAYS, MAX_STEPS, 3),
            valid.reshape(N_RAYS, MAX_STEPS) != 0,
            jnp.int32(MAX_STEPS),
            z.reshape(N_RAYS, MAX_STEPS))
